# baseline (speedup 1.0000x reference)
import functools

import jax
import jax.numpy as jnp
from jax.experimental import pallas as pl
from jax.experimental.pallas import tpu as pltpu

_ROWS = 1024


def _router_kernel(task_id_ref, mode_id_ref, x_ref, w_ref, tb_ref, mb_ref,
                   idx_ref, wgt_ref, acc_ref, *, B, S):
    c = pl.program_id(0)
    nc = pl.num_programs(0)
    D = x_ref.shape[1]
    cpb = S // _ROWS

    @pl.when(c == 0)
    def _init():
        acc_ref[:] = jnp.zeros_like(acc_ref)

    t = x_ref[:].reshape(_ROWS // 8, 8, D)
    v8 = jnp.sum(t, axis=0)
    pl8 = jax.lax.dot_general(
        v8, w_ref[:], (((1,), (1,)), ((), ())),
        preferred_element_type=jnp.float32)
    bb = c // cpb
    for b in range(B):
        @pl.when(bb == b)
        def _acc_b():
            acc_ref[pl.ds(8 * b, 8), :] += pl8

    @pl.when(c == nc - 1)
    def _finish():
        E = w_ref.shape[0]
        T = tb_ref.shape[0]
        M = mb_ref.shape[0]

        acc = acc_ref[:]
        logits = jnp.concatenate(
            [jnp.sum(acc[8 * b:8 * b + 8], axis=0, keepdims=True)
             for b in range(B)], axis=0) * (1.0 / S)

        t_iota = jax.lax.broadcasted_iota(jnp.int32, (1, T), 1)
        m_iota = jax.lax.broadcasted_iota(jnp.int32, (1, M), 1)
        oh_t = jnp.concatenate(
            [(t_iota == task_id_ref[b]).astype(jnp.float32) for b in range(B)],
            axis=0)
        oh_m = jnp.concatenate(
            [(m_iota == mode_id_ref[b]).astype(jnp.float32) for b in range(B)],
            axis=0)
        lg = logits + oh_t @ tb_ref[:] + oh_m @ mb_ref[:]

        e_iota = jax.lax.broadcasted_iota(jnp.int32, (B, E), 1)
        m1 = jnp.max(lg, axis=1, keepdims=True)
        i1 = jnp.min(jnp.where(lg == m1, e_iota, E), axis=1, keepdims=True)
        masked = jnp.where(e_iota == i1, -jnp.inf, lg)
        m2 = jnp.max(masked, axis=1, keepdims=True)
        i2 = jnp.min(jnp.where(masked == m2, e_iota, E), axis=1,
                     keepdims=True)

        idx_ref[:] = jnp.concatenate([i1, i2], axis=1)
        r = jnp.exp(m2 - m1)
        w1 = 1.0 / (1.0 + r)
        wgt_ref[:] = jnp.concatenate([w1, 1.0 - w1], axis=1)


@jax.jit
def _impl(x, task_id, mode_id, W, task_bias, mode_bias):
    B, S, D = x.shape
    nc = (B * S) // _ROWS

    idx, wgt = pl.pallas_call(
        functools.partial(_router_kernel, B=B, S=S),
        grid=(nc,),
        in_specs=[
            pl.BlockSpec(memory_space=pltpu.SMEM),
            pl.BlockSpec(memory_space=pltpu.SMEM),
            pl.BlockSpec((_ROWS, D), lambda c: (c, 0)),
            pl.BlockSpec(W.shape, lambda c: (0, 0)),
            pl.BlockSpec(task_bias.shape, lambda c: (0, 0)),
            pl.BlockSpec(mode_bias.shape, lambda c: (0, 0)),
        ],
        out_specs=[
            pl.BlockSpec((B, 2), lambda c: (0, 0)),
            pl.BlockSpec((B, 2), lambda c: (0, 0)),
        ],
        out_shape=[
            jax.ShapeDtypeStruct((B, 2), jnp.int32),
            jax.ShapeDtypeStruct((B, 2), jnp.float32),
        ],
        scratch_shapes=[pltpu.VMEM((8 * B, W.shape[0]), jnp.float32)],
        compiler_params=pltpu.CompilerParams(
            dimension_semantics=("arbitrary",)),
    )(task_id.astype(jnp.int32), mode_id.astype(jnp.int32),
      x.reshape(B * S, D), W, task_bias, mode_bias)
    return idx, wgt


def kernel(x, task_id, mode_id, W, task_bias, mode_bias):
    return _impl(x, task_id, mode_id, W, task_bias, mode_bias)


# speedup vs baseline: 1.0498x; 1.0498x over previous
import functools

import jax
import jax.numpy as jnp
from jax.experimental import pallas as pl
from jax.experimental.pallas import tpu as pltpu

_ROWS = 1024


def _router_kernel(task_id_ref, mode_id_ref, x_ref, w_ref, tb_ref, mb_ref,
                   idx_ref, wgt_ref, acc_ref, *, B, S):
    c = pl.program_id(0)
    nc = pl.num_programs(0)
    D = x_ref.shape[1]
    cpb = S // _ROWS

    @pl.when(c == 0)
    def _init():
        acc_ref[:] = jnp.zeros_like(acc_ref)

    t = x_ref[:].reshape(_ROWS // 8, 8, D)
    v8 = jnp.sum(t, axis=0)
    bb = c // cpb
    for b in range(B):
        @pl.when(bb == b)
        def _acc_b():
            acc_ref[pl.ds(8 * b, 8), :] += v8

    @pl.when(c == nc - 1)
    def _finish():
        E = w_ref.shape[0]
        T = tb_ref.shape[0]
        M = mb_ref.shape[0]

        acc = acc_ref[:]
        pooled = jnp.concatenate(
            [jnp.sum(acc[8 * b:8 * b + 8], axis=0, keepdims=True)
             for b in range(B)], axis=0) * (1.0 / S)
        logits = jax.lax.dot_general(
            pooled, w_ref[:], (((1,), (1,)), ((), ())),
            preferred_element_type=jnp.float32)

        t_iota = jax.lax.broadcasted_iota(jnp.int32, (1, T), 1)
        m_iota = jax.lax.broadcasted_iota(jnp.int32, (1, M), 1)
        oh_t = jnp.concatenate(
            [(t_iota == task_id_ref[b]).astype(jnp.float32) for b in range(B)],
            axis=0)
        oh_m = jnp.concatenate(
            [(m_iota == mode_id_ref[b]).astype(jnp.float32) for b in range(B)],
            axis=0)
        lg = logits + oh_t @ tb_ref[:] + oh_m @ mb_ref[:]

        e_iota = jax.lax.broadcasted_iota(jnp.int32, (B, E), 1)
        m1 = jnp.max(lg, axis=1, keepdims=True)
        i1 = jnp.min(jnp.where(lg == m1, e_iota, E), axis=1, keepdims=True)
        masked = jnp.where(e_iota == i1, -jnp.inf, lg)
        m2 = jnp.max(masked, axis=1, keepdims=True)
        i2 = jnp.min(jnp.where(masked == m2, e_iota, E), axis=1,
                     keepdims=True)

        idx_ref[:] = jnp.concatenate([i1, i2], axis=1)
        r = jnp.exp(m2 - m1)
        w1 = 1.0 / (1.0 + r)
        wgt_ref[:] = jnp.concatenate([w1, 1.0 - w1], axis=1)


@jax.jit
def _impl(x, task_id, mode_id, W, task_bias, mode_bias):
    B, S, D = x.shape
    nc = (B * S) // _ROWS

    idx, wgt = pl.pallas_call(
        functools.partial(_router_kernel, B=B, S=S),
        grid=(nc,),
        in_specs=[
            pl.BlockSpec(memory_space=pltpu.SMEM),
            pl.BlockSpec(memory_space=pltpu.SMEM),
            pl.BlockSpec((_ROWS, D), lambda c: (c, 0)),
            pl.BlockSpec(W.shape, lambda c: (0, 0)),
            pl.BlockSpec(task_bias.shape, lambda c: (0, 0)),
            pl.BlockSpec(mode_bias.shape, lambda c: (0, 0)),
        ],
        out_specs=[
            pl.BlockSpec((B, 2), lambda c: (0, 0)),
            pl.BlockSpec((B, 2), lambda c: (0, 0)),
        ],
        out_shape=[
            jax.ShapeDtypeStruct((B, 2), jnp.int32),
            jax.ShapeDtypeStruct((B, 2), jnp.float32),
        ],
        scratch_shapes=[pltpu.VMEM((8 * B, D), jnp.float32)],
        compiler_params=pltpu.CompilerParams(
            dimension_semantics=("arbitrary",)),
    )(task_id.astype(jnp.int32), mode_id.astype(jnp.int32),
      x.reshape(B * S, D), W, task_bias, mode_bias)
    return idx, wgt


def kernel(x, task_id, mode_id, W, task_bias, mode_bias):
    return _impl(x, task_id, mode_id, W, task_bias, mode_bias)
